# Initial kernel scaffold; baseline (speedup 1.0000x reference)
import jax
import jax.numpy as jnp
from jax.experimental import pallas as pl


def kernel(x):
    raise NotImplementedError("write your pallas kernel here")


# speedup vs baseline: 5.1655x; 5.1655x over previous
import jax
import jax.numpy as jnp
import numpy as np
from jax import lax
from jax.experimental import pallas as pl
from jax.experimental.pallas import tpu as pltpu
from jax.experimental.pallas import tpu_sc as plsc

_BETA = 0.25
_T = 4
_B = 32
_N = 384 * 16 * 16
_K = _N // 2
_L = 16
_CH = 16384
_NCH = _N // _CH
_HI = np.uint32(0x80000000)
_ALL1 = np.uint32(0xFFFFFFFF)


def _sort_key(m):
    bu = plsc.bitcast(m, jnp.uint32)
    flip = jnp.where(bu >= _HI, _ALL1, _HI)
    return bu ^ flip


def _scan_bins(hist, k_rem, lanes):
    def jbody(jj, carry):
        acc, found, vstar, above = carry
        j = 15 - jj

        def lbody(l, cv):
            return cv + hist[pl.ds(l * 256 + j * 16, 16)]

        c_vec = lax.fori_loop(0, 16, lbody, jnp.zeros((16,), jnp.int32))
        tot = jnp.sum(c_vec)
        s_local = lax.rev(plsc.cumsum(lax.rev(c_vec, (0,))), (0,))
        hit = jnp.logical_and(found == 0, acc + tot >= k_rem)
        mvec = (acc + s_local) >= k_rem
        pc = plsc.all_reduce_population_count(mvec)
        i0 = jnp.max(pc) - 1
        cv_at = jnp.sum(jnp.where(lanes == i0, c_vec, 0))
        sv_at = jnp.sum(jnp.where(lanes == i0, s_local, 0))
        above_j = acc + sv_at - cv_at
        vstar = jnp.where(hit, j * 16 + i0, vstar)
        above = jnp.where(hit, above_j, above)
        found = jnp.where(hit, 1, found)
        return (acc + tot, found, vstar, above)

    _, _, vstar, above = lax.fori_loop(
        0, 16, jbody, (jnp.int32(0), jnp.int32(0), jnp.int32(0), jnp.int32(0)))
    return vstar, above


def _lif_body(x_hbm, out_hbm, mem, hist, buf):
    c = lax.axis_index("c")
    s = lax.axis_index("s")
    b = s * 2 + c
    lanes = lax.iota(jnp.int32, _L)
    ones_i32 = jnp.ones((_L,), jnp.int32)
    zeros_f = jnp.zeros((_L,), jnp.float32)
    zeros_i = jnp.zeros((_L,), jnp.int32)

    def zmem(i, _):
        mem[pl.ds(i * _L, _L)] = zeros_f
        return 0

    lax.fori_loop(0, _N // _L, zmem, 0)

    def zhist(i, _):
        hist[pl.ds(i * _L, _L)] = zeros_i
        return 0

    for t in range(_T):
        lax.fori_loop(0, (16 * 256) // _L, zhist, 0)

        for ch in range(_NCH):
            pltpu.sync_copy(x_hbm.at[t, b, pl.ds(ch * _CH, _CH)], buf)

            def upd(i, _, ch=ch):
                gidx = pl.ds(ch * _CH + i * _L, _L)
                m = jnp.float32(_BETA) * mem[gidx] + buf[pl.ds(i * _L, _L)]
                mem[gidx] = m
                key = _sort_key(m)
                byte = (key >> jnp.uint32(24)).astype(jnp.int32)
                plsc.addupdate_scatter(hist, [lanes * 256 + byte], ones_i32)
                return 0

            lax.fori_loop(0, _CH // _L, upd, 0)

        k_rem = jnp.int32(_K)
        prefix = jnp.uint32(0)
        for pi, shift in enumerate((24, 16, 8, 0)):
            if pi > 0:
                lax.fori_loop(0, (16 * 256) // _L, zhist, 0)
                hi_sh = jnp.uint32(shift + 8)
                pref_hi = prefix >> hi_sh

                def hpass(i, _, shift=shift, hi_sh=hi_sh, pref_hi=pref_hi):
                    key = _sort_key(mem[pl.ds(i * _L, _L)])
                    match = (key >> hi_sh) == pref_hi
                    byte = ((key >> jnp.uint32(shift)) & jnp.uint32(0xFF)
                            ).astype(jnp.int32)
                    plsc.addupdate_scatter(
                        hist, [lanes * 256 + byte], ones_i32, mask=match)
                    return 0

                lax.fori_loop(0, _N // _L, hpass, 0)
            vstar, above = _scan_bins(hist, k_rem, lanes)
            k_rem = k_rem - above
            prefix = prefix | (vstar.astype(jnp.uint32) << jnp.uint32(shift))

        thr_bits = jnp.where(prefix >= _HI, prefix ^ _HI, prefix ^ _ALL1)
        thr = plsc.bitcast(jnp.broadcast_to(thr_bits, (_L,)), jnp.float32)

        for ch in range(_NCH):

            def spk(i, _, ch=ch):
                gidx = pl.ds(ch * _CH + i * _L, _L)
                m = mem[gidx]
                ge = m >= thr
                buf[pl.ds(i * _L, _L)] = jnp.where(ge, jnp.float32(1.0),
                                                   jnp.float32(0.0))
                mem[gidx] = jnp.where(ge, jnp.float32(0.0), m)
                return 0

            lax.fori_loop(0, _CH // _L, spk, 0)
            pltpu.sync_copy(buf, out_hbm.at[t, b, pl.ds(ch * _CH, _CH)])


def kernel(x):
    lif = pl.kernel(
        _lif_body,
        out_type=jax.ShapeDtypeStruct((_T, _B, _N), jnp.float32),
        mesh=plsc.VectorSubcoreMesh(core_axis_name="c", subcore_axis_name="s"),
        compiler_params=pltpu.CompilerParams(needs_layout_passes=False),
        scratch_types=[
            pltpu.VMEM((_N,), jnp.float32),
            pltpu.VMEM((16 * 256,), jnp.int32),
            pltpu.VMEM((_CH,), jnp.float32),
        ],
    )
    y = lif(x.reshape(_T, _B, _N))
    return y.reshape(x.shape)


# speedup vs baseline: 5.7004x; 1.1035x over previous
import jax
import jax.numpy as jnp
import numpy as np
from jax import lax
from jax.experimental import pallas as pl
from jax.experimental.pallas import tpu as pltpu
from jax.experimental.pallas import tpu_sc as plsc

_BETA = 0.25
_T = 4
_B = 32
_N = 384 * 16 * 16
_K = _N // 2
_L = 16
_CH = 16384
_NCH = _N // _CH
_HI = np.uint32(0x80000000)
_ALL1 = np.uint32(0xFFFFFFFF)


def _sort_key(m):
    bu = plsc.bitcast(m, jnp.uint32)
    flip = jnp.where(bu >= _HI, _ALL1, _HI)
    return bu ^ flip


def _scan_bins(hist, k_rem, lanes):
    def jbody(jj, carry):
        acc, found, vstar, above = carry
        j = 15 - jj

        def lbody(l, cv):
            return cv + hist[pl.ds(l * 256 + j * 16, 16)]

        c_vec = lax.fori_loop(0, 16, lbody, jnp.zeros((16,), jnp.int32))
        tot = jnp.sum(c_vec)
        s_local = lax.rev(plsc.cumsum(lax.rev(c_vec, (0,))), (0,))
        hit = jnp.logical_and(found == 0, acc + tot >= k_rem)
        mvec = (acc + s_local) >= k_rem
        pc = plsc.all_reduce_population_count(mvec)
        i0 = jnp.max(pc) - 1
        cv_at = jnp.sum(jnp.where(lanes == i0, c_vec, 0))
        sv_at = jnp.sum(jnp.where(lanes == i0, s_local, 0))
        above_j = acc + sv_at - cv_at
        vstar = jnp.where(hit, j * 16 + i0, vstar)
        above = jnp.where(hit, above_j, above)
        found = jnp.where(hit, 1, found)
        return (acc + tot, found, vstar, above)

    _, _, vstar, above = lax.fori_loop(
        0, 16, jbody, (jnp.int32(0), jnp.int32(0), jnp.int32(0), jnp.int32(0)))
    return vstar, above


def _lif_body(x_hbm, out_hbm, mem, hist, buf):
    c = lax.axis_index("c")
    s = lax.axis_index("s")
    b = s * 2 + c
    lanes = lax.iota(jnp.int32, _L)
    ones_i32 = jnp.ones((_L,), jnp.int32)
    zeros_f = jnp.zeros((_L,), jnp.float32)
    zeros_i = jnp.zeros((_L,), jnp.int32)

    def zmem(i, _):
        mem[pl.ds(i * _L, _L)] = zeros_f
        return 0

    lax.fori_loop(0, _N // _L, zmem, 0, unroll=8)

    def zhist(i, _):
        hist[pl.ds(i * _L, _L)] = zeros_i
        return 0

    for t in range(_T):
        lax.fori_loop(0, (16 * 256) // _L, zhist, 0, unroll=8)

        for ch in range(_NCH):
            pltpu.sync_copy(x_hbm.at[t, b, pl.ds(ch * _CH, _CH)], buf)

            def upd(i, _, ch=ch):
                gidx = pl.ds(ch * _CH + i * _L, _L)
                m = jnp.float32(_BETA) * mem[gidx] + buf[pl.ds(i * _L, _L)]
                mem[gidx] = m
                key = _sort_key(m)
                byte = (key >> jnp.uint32(24)).astype(jnp.int32)
                plsc.addupdate_scatter(hist, [lanes * 256 + byte], ones_i32)
                return 0

            lax.fori_loop(0, _CH // _L, upd, 0, unroll=8)

        k_rem = jnp.int32(_K)
        prefix = jnp.uint32(0)
        for pi, shift in enumerate((24, 16, 8, 0)):
            if pi > 0:
                lax.fori_loop(0, (16 * 256) // _L, zhist, 0, unroll=8)
                hi_sh = jnp.uint32(shift + 8)
                pref_hi = prefix >> hi_sh

                def hpass(i, _, shift=shift, hi_sh=hi_sh, pref_hi=pref_hi):
                    key = _sort_key(mem[pl.ds(i * _L, _L)])
                    match = (key >> hi_sh) == pref_hi
                    byte = ((key >> jnp.uint32(shift)) & jnp.uint32(0xFF)
                            ).astype(jnp.int32)
                    plsc.addupdate_scatter(
                        hist, [lanes * 256 + byte], ones_i32, mask=match)
                    return 0

                lax.fori_loop(0, _N // _L, hpass, 0, unroll=8)
            vstar, above = _scan_bins(hist, k_rem, lanes)
            k_rem = k_rem - above
            prefix = prefix | (vstar.astype(jnp.uint32) << jnp.uint32(shift))

        thr_bits = jnp.where(prefix >= _HI, prefix ^ _HI, prefix ^ _ALL1)
        thr = plsc.bitcast(jnp.broadcast_to(thr_bits, (_L,)), jnp.float32)

        for ch in range(_NCH):

            def spk(i, _, ch=ch):
                gidx = pl.ds(ch * _CH + i * _L, _L)
                m = mem[gidx]
                ge = m >= thr
                buf[pl.ds(i * _L, _L)] = jnp.where(ge, jnp.float32(1.0),
                                                   jnp.float32(0.0))
                mem[gidx] = jnp.where(ge, jnp.float32(0.0), m)
                return 0

            lax.fori_loop(0, _CH // _L, spk, 0, unroll=8)
            pltpu.sync_copy(buf, out_hbm.at[t, b, pl.ds(ch * _CH, _CH)])


def kernel(x):
    lif = pl.kernel(
        _lif_body,
        out_type=jax.ShapeDtypeStruct((_T, _B, _N), jnp.float32),
        mesh=plsc.VectorSubcoreMesh(core_axis_name="c", subcore_axis_name="s"),
        compiler_params=pltpu.CompilerParams(needs_layout_passes=False),
        scratch_types=[
            pltpu.VMEM((_N,), jnp.float32),
            pltpu.VMEM((16 * 256,), jnp.int32),
            pltpu.VMEM((_CH,), jnp.float32),
        ],
    )
    y = lif(x.reshape(_T, _B, _N))
    return y.reshape(x.shape)


# speedup vs baseline: 6.8209x; 1.1966x over previous
import jax
import jax.numpy as jnp
import numpy as np
from jax import lax
from jax.experimental import pallas as pl
from jax.experimental.pallas import tpu as pltpu
from jax.experimental.pallas import tpu_sc as plsc

_BETA = 0.25
_T = 4
_B = 32
_N = 384 * 16 * 16
_K = _N // 2
_L = 16
_CH = 16384
_NCH = _N // _CH
_HI = np.uint32(0x80000000)
_ALL1 = np.uint32(0xFFFFFFFF)
_PASSES = ((20, 12), (10, 10), (0, 10))


def _sort_key(m):
    bu = plsc.bitcast(m, jnp.uint32)
    flip = jnp.where(bu >= _HI, _ALL1, _HI)
    return bu ^ flip


def _scan_bins(hist, k_rem, lanes, nbins):
    groups = nbins // _L

    def jbody(jj, carry):
        acc, found, vstar, above = carry
        j = groups - 1 - jj
        c_vec = hist[pl.ds(j * _L, _L)]
        tot = jnp.sum(c_vec)
        s_local = lax.rev(plsc.cumsum(lax.rev(c_vec, (0,))), (0,))
        hit = jnp.logical_and(found == 0, acc + tot >= k_rem)
        mvec = (acc + s_local) >= k_rem
        pc = plsc.all_reduce_population_count(mvec)
        i0 = jnp.max(pc) - 1
        cv_at = jnp.sum(jnp.where(lanes == i0, c_vec, 0))
        sv_at = jnp.sum(jnp.where(lanes == i0, s_local, 0))
        above_j = acc + sv_at - cv_at
        vstar = jnp.where(hit, j * _L + i0, vstar)
        above = jnp.where(hit, above_j, above)
        found = jnp.where(hit, 1, found)
        return (acc + tot, found, vstar, above)

    _, _, vstar, above = lax.fori_loop(
        0, groups, jbody,
        (jnp.int32(0), jnp.int32(0), jnp.int32(0), jnp.int32(0)))
    return vstar, above


def _lif_body(x_hbm, out_hbm, mem, hist, buf):
    c = lax.axis_index("c")
    s = lax.axis_index("s")
    b = s * 2 + c
    lanes = lax.iota(jnp.int32, _L)
    ones_i32 = jnp.ones((_L,), jnp.int32)
    zeros_f = jnp.zeros((_L,), jnp.float32)
    zeros_i = jnp.zeros((_L,), jnp.int32)

    def zmem(i, _):
        mem[pl.ds(i * _L, _L)] = zeros_f
        return 0

    lax.fori_loop(0, _N // _L, zmem, 0, unroll=8)

    def zhist(i, _):
        hist[pl.ds(i * _L, _L)] = zeros_i
        return 0

    sh0, w0 = _PASSES[0]
    for t in range(_T):
        lax.fori_loop(0, (1 << w0) // _L, zhist, 0, unroll=8)

        for ch in range(_NCH):
            pltpu.sync_copy(x_hbm.at[t, b, pl.ds(ch * _CH, _CH)], buf)

            def upd(i, _, ch=ch):
                gidx = pl.ds(ch * _CH + i * _L, _L)
                m = jnp.float32(_BETA) * mem[gidx] + buf[pl.ds(i * _L, _L)]
                mem[gidx] = m
                key = _sort_key(m)
                digit = (key >> jnp.uint32(sh0)).astype(jnp.int32)
                plsc.addupdate_scatter(hist, [digit], ones_i32)
                return 0

            lax.fori_loop(0, _CH // _L, upd, 0, unroll=8)

        k_rem = jnp.int32(_K)
        prefix = jnp.uint32(0)
        for pi, (shift, width) in enumerate(_PASSES):
            if pi > 0:
                lax.fori_loop(0, (1 << width) // _L, zhist, 0, unroll=8)
                hi_sh = jnp.uint32(shift + width)
                pref_hi = prefix >> hi_sh
                dmask = jnp.uint32((1 << width) - 1)

                def hpass(i, _, shift=shift, hi_sh=hi_sh, pref_hi=pref_hi,
                          dmask=dmask):
                    key = _sort_key(mem[pl.ds(i * _L, _L)])
                    match = (key >> hi_sh) == pref_hi
                    digit = ((key >> jnp.uint32(shift)) & dmask
                             ).astype(jnp.int32)
                    plsc.addupdate_scatter(hist, [digit], ones_i32,
                                           mask=match)
                    return 0

                lax.fori_loop(0, _N // _L, hpass, 0, unroll=8)
            vstar, above = _scan_bins(hist, k_rem, lanes, 1 << width)
            k_rem = k_rem - above
            prefix = prefix | (vstar.astype(jnp.uint32) << jnp.uint32(shift))

        thr_bits = jnp.where(prefix >= _HI, prefix ^ _HI, prefix ^ _ALL1)
        thr = plsc.bitcast(jnp.broadcast_to(thr_bits, (_L,)), jnp.float32)

        for ch in range(_NCH):

            def spk(i, _, ch=ch):
                gidx = pl.ds(ch * _CH + i * _L, _L)
                m = mem[gidx]
                ge = m >= thr
                buf[pl.ds(i * _L, _L)] = jnp.where(ge, jnp.float32(1.0),
                                                   jnp.float32(0.0))
                mem[gidx] = jnp.where(ge, jnp.float32(0.0), m)
                return 0

            lax.fori_loop(0, _CH // _L, spk, 0, unroll=8)
            pltpu.sync_copy(buf, out_hbm.at[t, b, pl.ds(ch * _CH, _CH)])


def kernel(x):
    lif = pl.kernel(
        _lif_body,
        out_type=jax.ShapeDtypeStruct((_T, _B, _N), jnp.float32),
        mesh=plsc.VectorSubcoreMesh(core_axis_name="c", subcore_axis_name="s"),
        compiler_params=pltpu.CompilerParams(needs_layout_passes=False),
        scratch_types=[
            pltpu.VMEM((_N,), jnp.float32),
            pltpu.VMEM((1 << _PASSES[0][1],), jnp.int32),
            pltpu.VMEM((_CH,), jnp.float32),
        ],
    )
    y = lif(x.reshape(_T, _B, _N))
    return y.reshape(x.shape)


# speedup vs baseline: 11.8460x; 1.7367x over previous
import jax
import jax.numpy as jnp
import numpy as np
from jax import lax
from jax.experimental import pallas as pl
from jax.experimental.pallas import tpu as pltpu
from jax.experimental.pallas import tpu_sc as plsc

_BETA = 0.25
_T = 4
_B = 32
_N = 384 * 16 * 16
_K = _N // 2
_L = 16
_CH = 16384
_NCH = _N // _CH
_HI = np.uint32(0x80000000)
_ALL1 = np.uint32(0xFFFFFFFF)
_PASSES = ((20, 12), (10, 10), (0, 10))


def _sort_key(m):
    bu = plsc.bitcast(m, jnp.uint32)
    flip = jnp.where(bu >= _HI, _ALL1, _HI)
    return bu ^ flip


def _scan_bins(hist, k_rem, lanes, nbins):
    groups = nbins // _L

    def jbody(jj, carry):
        acc, found, vstar, above = carry
        j = groups - 1 - jj
        c_vec = hist[pl.ds(j * _L, _L)]
        tot = jnp.sum(c_vec)
        s_local = lax.rev(plsc.cumsum(lax.rev(c_vec, (0,))), (0,))
        hit = jnp.logical_and(found == 0, acc + tot >= k_rem)
        mvec = (acc + s_local) >= k_rem
        pc = plsc.all_reduce_population_count(mvec)
        i0 = jnp.max(pc) - 1
        cv_at = jnp.sum(jnp.where(lanes == i0, c_vec, 0))
        sv_at = jnp.sum(jnp.where(lanes == i0, s_local, 0))
        above_j = acc + sv_at - cv_at
        vstar = jnp.where(hit, j * _L + i0, vstar)
        above = jnp.where(hit, above_j, above)
        found = jnp.where(hit, 1, found)
        return (acc + tot, found, vstar, above)

    _, _, vstar, above = lax.fori_loop(
        0, groups, jbody,
        (jnp.int32(0), jnp.int32(0), jnp.int32(0), jnp.int32(0)))
    return vstar, above


def _lif_body(x_hbm, out_hbm, mem, hist, buf):
    c = lax.axis_index("c")
    s = lax.axis_index("s")
    b = s * 2 + c
    lanes = lax.iota(jnp.int32, _L)
    ones_i32 = jnp.ones((_L,), jnp.int32)
    zeros_f = jnp.zeros((_L,), jnp.float32)
    zeros_i = jnp.zeros((_L,), jnp.int32)

    @plsc.parallel_loop(0, _N // _L, unroll=8)
    def _(i):
        mem[pl.ds(i * _L, _L)] = zeros_f

    def zhist(nbins):
        @plsc.parallel_loop(0, nbins // _L, unroll=8)
        def _(i):
            hist[pl.ds(i * _L, _L)] = zeros_i

    sh0, w0 = _PASSES[0]
    for t in range(_T):
        zhist(1 << w0)

        for ch in range(_NCH):
            pltpu.sync_copy(x_hbm.at[t, b, pl.ds(ch * _CH, _CH)], buf)

            @plsc.parallel_loop(0, _CH // _L, unroll=8)
            def _(i, ch=ch):
                gidx = pl.ds(ch * _CH + i * _L, _L)
                m = jnp.float32(_BETA) * mem[gidx] + buf[pl.ds(i * _L, _L)]
                mem[gidx] = m
                key = _sort_key(m)
                digit = (key >> jnp.uint32(sh0)).astype(jnp.int32)
                plsc.addupdate_scatter(hist, [digit], ones_i32)

        k_rem = jnp.int32(_K)
        prefix = jnp.uint32(0)
        for pi, (shift, width) in enumerate(_PASSES):
            if pi > 0:
                zhist(1 << width)
                hi_sh = jnp.uint32(shift + width)
                pref_hi = prefix >> hi_sh
                dmask = jnp.uint32((1 << width) - 1)

                @plsc.parallel_loop(0, _N // _L, unroll=8)
                def _(i, shift=shift, hi_sh=hi_sh, pref_hi=pref_hi,
                      dmask=dmask):
                    key = _sort_key(mem[pl.ds(i * _L, _L)])
                    match = (key >> hi_sh) == pref_hi
                    digit = ((key >> jnp.uint32(shift)) & dmask
                             ).astype(jnp.int32)
                    plsc.addupdate_scatter(hist, [digit], ones_i32,
                                           mask=match)
            vstar, above = _scan_bins(hist, k_rem, lanes, 1 << width)
            k_rem = k_rem - above
            prefix = prefix | (vstar.astype(jnp.uint32) << jnp.uint32(shift))

        thr_bits = jnp.where(prefix >= _HI, prefix ^ _HI, prefix ^ _ALL1)
        thr = plsc.bitcast(jnp.broadcast_to(thr_bits, (_L,)), jnp.float32)

        for ch in range(_NCH):

            @plsc.parallel_loop(0, _CH // _L, unroll=8)
            def _(i, ch=ch):
                gidx = pl.ds(ch * _CH + i * _L, _L)
                m = mem[gidx]
                ge = m >= thr
                buf[pl.ds(i * _L, _L)] = jnp.where(ge, jnp.float32(1.0),
                                                   jnp.float32(0.0))
                mem[gidx] = jnp.where(ge, jnp.float32(0.0), m)
            pltpu.sync_copy(buf, out_hbm.at[t, b, pl.ds(ch * _CH, _CH)])


def kernel(x):
    lif = pl.kernel(
        _lif_body,
        out_type=jax.ShapeDtypeStruct((_T, _B, _N), jnp.float32),
        mesh=plsc.VectorSubcoreMesh(core_axis_name="c", subcore_axis_name="s"),
        compiler_params=pltpu.CompilerParams(needs_layout_passes=False),
        scratch_types=[
            pltpu.VMEM((_N,), jnp.float32),
            pltpu.VMEM((1 << _PASSES[0][1],), jnp.int32),
            pltpu.VMEM((_CH,), jnp.float32),
        ],
    )
    y = lif(x.reshape(_T, _B, _N))
    return y.reshape(x.shape)


# speedup vs baseline: 12.1181x; 1.0230x over previous
import jax
import jax.numpy as jnp
import numpy as np
from jax import lax
from jax.experimental import pallas as pl
from jax.experimental.pallas import tpu as pltpu
from jax.experimental.pallas import tpu_sc as plsc

_BETA = 0.25
_T = 4
_B = 32
_N = 384 * 16 * 16
_K = _N // 2
_L = 16
_CH = 8192
_NCH = _N // _CH
_HI = np.uint32(0x80000000)
_ALL1 = np.uint32(0xFFFFFFFF)
_PASSES = ((20, 12), (10, 10), (0, 10))


def _sort_key(m):
    bu = plsc.bitcast(m, jnp.uint32)
    flip = jnp.where(bu >= _HI, _ALL1, _HI)
    return bu ^ flip


def _scan_bins(hist, k_rem, lanes, nbins):
    groups = nbins // _L

    def jbody(jj, carry):
        acc, found, vstar, above = carry
        j = groups - 1 - jj
        c_vec = hist[pl.ds(j * _L, _L)]
        tot = jnp.sum(c_vec)
        s_local = lax.rev(plsc.cumsum(lax.rev(c_vec, (0,))), (0,))
        hit = jnp.logical_and(found == 0, acc + tot >= k_rem)
        mvec = (acc + s_local) >= k_rem
        pc = plsc.all_reduce_population_count(mvec)
        i0 = jnp.max(pc) - 1
        cv_at = jnp.sum(jnp.where(lanes == i0, c_vec, 0))
        sv_at = jnp.sum(jnp.where(lanes == i0, s_local, 0))
        above_j = acc + sv_at - cv_at
        vstar = jnp.where(hit, j * _L + i0, vstar)
        above = jnp.where(hit, above_j, above)
        found = jnp.where(hit, 1, found)
        return (acc + tot, found, vstar, above)

    _, _, vstar, above = lax.fori_loop(
        0, groups, jbody,
        (jnp.int32(0), jnp.int32(0), jnp.int32(0), jnp.int32(0)))
    return vstar, above


def _lif_body(x_hbm, out_hbm, mem, hist, xbuf, sbuf):
    c = lax.axis_index("c")
    s = lax.axis_index("s")
    b = s * 2 + c
    lanes = lax.iota(jnp.int32, _L)
    ones_i32 = jnp.ones((_L,), jnp.int32)
    zeros_i = jnp.zeros((_L,), jnp.int32)

    def zhist(nbins):
        @plsc.parallel_loop(0, nbins // _L, unroll=8)
        def _(i):
            hist[pl.ds(i * _L, _L)] = zeros_i

    sh0, w0 = _PASSES[0]
    thr = jnp.broadcast_to(jnp.float32(0.0), (_L,))
    for t in range(_T):
        zhist(1 << w0)

        for ch in range(_NCH):
            pltpu.sync_copy(x_hbm.at[t, b, pl.ds(ch * _CH, _CH)], xbuf)

            if t == 0:
                @plsc.parallel_loop(0, _CH // _L, unroll=8)
                def _(i, ch=ch):
                    gidx = pl.ds(ch * _CH + i * _L, _L)
                    m = xbuf[pl.ds(i * _L, _L)]
                    mem[gidx] = m
                    key = _sort_key(m)
                    digit = (key >> jnp.uint32(sh0)).astype(jnp.int32)
                    plsc.addupdate_scatter(hist, [digit], ones_i32)
            else:
                @plsc.parallel_loop(0, _CH // _L, unroll=8)
                def _(i, ch=ch, thr=thr):
                    gidx = pl.ds(ch * _CH + i * _L, _L)
                    m = mem[gidx]
                    ge = m >= thr
                    sbuf[pl.ds(i * _L, _L)] = jnp.where(
                        ge, jnp.float32(1.0), jnp.float32(0.0))
                    m = (jnp.float32(_BETA)
                         * jnp.where(ge, jnp.float32(0.0), m)
                         + xbuf[pl.ds(i * _L, _L)])
                    mem[gidx] = m
                    key = _sort_key(m)
                    digit = (key >> jnp.uint32(sh0)).astype(jnp.int32)
                    plsc.addupdate_scatter(hist, [digit], ones_i32)
                pltpu.sync_copy(sbuf,
                                out_hbm.at[t - 1, b, pl.ds(ch * _CH, _CH)])

        k_rem = jnp.int32(_K)
        prefix = jnp.uint32(0)
        for pi, (shift, width) in enumerate(_PASSES):
            if pi > 0:
                zhist(1 << width)
                hi_sh = jnp.uint32(shift + width)
                pref_hi = prefix >> hi_sh
                dmask = jnp.uint32((1 << width) - 1)

                @plsc.parallel_loop(0, _N // _L, unroll=8)
                def _(i, shift=shift, hi_sh=hi_sh, pref_hi=pref_hi,
                      dmask=dmask):
                    key = _sort_key(mem[pl.ds(i * _L, _L)])
                    match = (key >> hi_sh) == pref_hi
                    digit = ((key >> jnp.uint32(shift)) & dmask
                             ).astype(jnp.int32)
                    plsc.addupdate_scatter(hist, [digit], ones_i32,
                                           mask=match)
            vstar, above = _scan_bins(hist, k_rem, lanes, 1 << width)
            k_rem = k_rem - above
            prefix = prefix | (vstar.astype(jnp.uint32) << jnp.uint32(shift))

        thr_bits = jnp.where(prefix >= _HI, prefix ^ _HI, prefix ^ _ALL1)
        thr = plsc.bitcast(jnp.broadcast_to(thr_bits, (_L,)), jnp.float32)

    for ch in range(_NCH):

        @plsc.parallel_loop(0, _CH // _L, unroll=8)
        def _(i, ch=ch, thr=thr):
            gidx = pl.ds(ch * _CH + i * _L, _L)
            m = mem[gidx]
            sbuf[pl.ds(i * _L, _L)] = jnp.where(
                m >= thr, jnp.float32(1.0), jnp.float32(0.0))
        pltpu.sync_copy(sbuf, out_hbm.at[_T - 1, b, pl.ds(ch * _CH, _CH)])


def kernel(x):
    lif = pl.kernel(
        _lif_body,
        out_type=jax.ShapeDtypeStruct((_T, _B, _N), jnp.float32),
        mesh=plsc.VectorSubcoreMesh(core_axis_name="c", subcore_axis_name="s"),
        compiler_params=pltpu.CompilerParams(needs_layout_passes=False),
        scratch_types=[
            pltpu.VMEM((_N,), jnp.float32),
            pltpu.VMEM((1 << _PASSES[0][1],), jnp.int32),
            pltpu.VMEM((_CH,), jnp.float32),
            pltpu.VMEM((_CH,), jnp.float32),
        ],
    )
    y = lif(x.reshape(_T, _B, _N))
    return y.reshape(x.shape)
